# baseline (speedup 1.0000x reference)
import functools

import numpy as np
import jax
import jax.numpy as jnp
from jax import lax
from jax.experimental import pallas as pl
from jax.experimental.pallas import tpu as pltpu
from jax.experimental.pallas import tpu_sc as plsc

_GRID = 256
_CANVAS = 2048
_NUM_DOTS = 4096
_PATCH = 15
_RADIUS = 2.0
_SIG1 = float(1.0 / (1.0 + np.exp(-1.0)))
_C1 = _SIG1 - 0.5
_BLK = 256
_NBLK = _CANVAS // _BLK
_KW = 64

_NT = 16
_RPT = _GRID // _NT
_EPT = _RPT * _GRID
_NROW = _EPT // 128
_NB = 1024
_HPAD = 1152


def _softmax_kernel(logits_ref, u_ref, soft_ref):
    g = -jnp.log(-jnp.log(u_ref[...]))
    y = logits_ref[...] + g
    y = y - jnp.max(y, axis=1, keepdims=True)
    ey = jnp.exp(y)
    soft_ref[...] = ey / jnp.sum(ey, axis=1, keepdims=True)


def _sc_iota16():
    return lax.broadcasted_iota(jnp.int32, (16,), 0)


def _hsum16(v):
    acc = v[0]
    for l in range(1, 16):
        acc = acc + v[l]
    return acc


def _sc_scan_hist(tmp_ref, csum_ref, need):
    above = jnp.float32(0.0)
    cid = jnp.int32(0)
    cabove = jnp.float32(0.0)
    for q in range(3, -1, -1):
        ch = csum_ref[16 * q:16 * (q + 1)]
        for l in range(15, -1, -1):
            v = ch[l]
            hit = jnp.logical_and(above < need, above + v >= need)
            cid = jnp.where(hit, 16 * q + l, cid)
            cabove = jnp.where(hit, above, cabove)
            above = above + v
    ch2 = tmp_ref[pl.ds(cid * 16, 16)]
    a = cabove
    b = jnp.int32(0)
    na = jnp.float32(0.0)
    cb = jnp.float32(0.0)
    for l in range(15, -1, -1):
        v = ch2[l]
        hit = jnp.logical_and(a < need, a + v >= need)
        b = jnp.where(hit, cid * 16 + l, b)
        na = jnp.where(hit, a, na)
        cb = jnp.where(hit, v, cb)
        a = a + v
    return b, na, cb


def _sc_body(soft_ref, mask_ref, vals_ref, keys_ref, ones_ref, tmp_ref,
             maskb_ref, stage_ref, pref16_ref, hist16_ref, cs_keys_ref,
             csum_ref, z80_ref, sem,
             sh_f1_ref, sh_f2_ref, sh_f3_ref, sh_cs_ref, sh_pref_ref):
    c = lax.axis_index("c")
    s = lax.axis_index("s")
    io16 = _sc_iota16()

    def ob(j, carry):
        ones_ref[pl.ds(j * 16, 16)] = jnp.ones((16,), jnp.float32)
        return carry

    lax.fori_loop(0, _EPT // 16, ob, 0)

    def ckb(j, carry):
        cs_keys_ref[pl.ds(j * 16, 16)] = (
            lax.shift_right_logical(j * 16 + io16, 4) + s * 80)
        return carry

    lax.fori_loop(0, _HPAD // 16, ckb, 0)
    for j in range(5):
        z80_ref[pl.ds(16 * j, 16)] = jnp.zeros((16,), jnp.float32)

    pltpu.sync_copy(soft_ref.at[pl.ds(s * _RPT, _RPT)], vals_ref)

    zeros = jnp.zeros((16,), jnp.float32)

    def zb(j, carry):
        tmp_ref[pl.ds(j * 16, 16)] = zeros
        return carry

    lax.fori_loop(0, _HPAD // 16, zb, 0)

    pltpu.sync_copy(tmp_ref, sh_f1_ref.at[pl.ds(s * _HPAD, _HPAD)])
    pltpu.sync_copy(tmp_ref, sh_f2_ref.at[pl.ds(s * _HPAD, _HPAD)])
    pltpu.sync_copy(tmp_ref, sh_f3_ref.at[pl.ds(s * _HPAD, _HPAD)])
    plsc.subcore_barrier()

    def bits_chunk(r, jj):
        return lax.bitcast_convert_type(vals_ref[r, 16 * jj:16 * (jj + 1)],
                                        jnp.int32)

    def do_level(sh_ref, key_fn):
        def kb(r, carry):
            for jj in range(16):
                i = r * 16 + jj
                keys_ref[pl.ds(i * 16, 16)] = (key_fn(bits_chunk(r, jj))
                                               + s * _HPAD)
            return carry

        lax.fori_loop(0, _RPT, kb, 0)

        pltpu.sync_copy(ones_ref, sh_ref.at[keys_ref], add=True)
        plsc.subcore_barrier()

        pltpu.sync_copy(sh_ref, hist16_ref)

        def fold(j, carry):
            acc = hist16_ref[pl.ds(j * 16, 16)]
            for t in range(1, _NT):
                acc = acc + hist16_ref[pl.ds(t * _HPAD + j * 16, 16)]
            tmp_ref[pl.ds(j * 16, 16)] = acc
            return carry

        lax.fori_loop(0, _HPAD // 16, fold, 0)

        pltpu.sync_copy(z80_ref, sh_cs_ref.at[pl.ds(s * 80, 80)])
        pltpu.sync_copy(tmp_ref, sh_cs_ref.at[cs_keys_ref], add=True)
        pltpu.sync_copy(sh_cs_ref.at[pl.ds(s * 80, 80)], csum_ref)

    trash = jnp.full((16,), _NB, jnp.int32)
    need1 = jnp.float32(float(_NUM_DOTS))

    do_level(sh_f1_ref, lambda b: lax.shift_right_logical(b, 20))
    b1, na1, _ = _sc_scan_hist(tmp_ref, csum_ref, need1)
    need2 = need1 - na1

    def key2(b):
        k1 = lax.shift_right_logical(b, 20)
        k2 = jnp.bitwise_and(lax.shift_right_logical(b, 10), 1023)
        return jnp.where(k1 == b1, k2, trash)

    do_level(sh_f2_ref, key2)
    b2, na2, _ = _sc_scan_hist(tmp_ref, csum_ref, need2)
    need3 = need2 - na2

    def key3(b):
        k1 = lax.shift_right_logical(b, 20)
        k2 = jnp.bitwise_and(lax.shift_right_logical(b, 10), 1023)
        k3 = jnp.bitwise_and(b, 1023)
        sel = jnp.logical_and(k1 == b1, k2 == b2)
        return jnp.where(sel, k3, trash)

    do_level(sh_f3_ref, key3)
    b3, na3, n_eq_tot = _sc_scan_hist(tmp_ref, csum_ref, need3)
    n_eq_needed = need3 - na3
    vstar = jnp.bitwise_or(
        jnp.bitwise_or(lax.shift_left(b1, 20), lax.shift_left(b2, 10)), b3)

    ties = n_eq_tot > n_eq_needed

    def count_eq(_):
        def eq_cnt_body(r, acc):
            for jj in range(16):
                eqf = jnp.where(bits_chunk(r, jj) == vstar, 1.0, 0.0)
                acc = acc + _hsum16(eqf)
            return acc

        return lax.fori_loop(0, _RPT, eq_cnt_body, jnp.float32(0.0))

    eq_local = lax.cond(ties, count_eq, lambda _: jnp.float32(0.0), 0)

    stage_ref[...] = jnp.zeros((16,), jnp.float32) + eq_local
    pltpu.sync_copy(stage_ref, sh_pref_ref.at[pl.ds(s * 16, 16)])
    plsc.subcore_barrier()
    pltpu.sync_copy(sh_pref_ref, pref16_ref)
    my_base = jnp.float32(0.0)
    for t in range(_NT):
        cnt_t = pref16_ref[pl.ds(t * 16, 16)][0]
        my_base = my_base + jnp.where(t < s, cnt_t, 0.0)

    def mk_simple(_):
        def body(r, carry):
            for jj in range(16):
                sel = bits_chunk(r, jj) >= vstar
                maskb_ref[r, 16 * jj:16 * (jj + 1)] = jnp.where(
                    sel, 1.0, 0.0)
            return carry

        lax.fori_loop(0, _RPT, body, 0)
        return 0

    def mk_ties(_):
        def body(r, run):
            prefix = run
            for jj in range(16):
                b = bits_chunk(r, jj)
                eqf = jnp.where(b == vstar, 1.0, 0.0)
                out = jnp.where(b > vstar, 1.0, 0.0)
                for l in range(16):
                    e = eqf[l]
                    takef = jnp.where(
                        jnp.logical_and(e > 0.5, prefix < n_eq_needed),
                        1.0, 0.0)
                    out = out + jnp.where(io16 == l, takef, 0.0)
                    prefix = prefix + e
                maskb_ref[r, 16 * jj:16 * (jj + 1)] = out
            return prefix

        lax.fori_loop(0, _RPT, body, my_base)
        return 0

    lax.cond(ties, mk_ties, mk_simple, 0)

    @pl.when(c == 0)
    def _write():
        pltpu.sync_copy(maskb_ref, mask_ref.at[pl.ds(s * _RPT, _RPT)])


def _sc_scratch_types():
    return [
        pltpu.VMEM((_RPT, _GRID), jnp.float32),
        pltpu.VMEM((_EPT,), jnp.int32),
        pltpu.VMEM((_EPT,), jnp.float32),
        pltpu.VMEM((_HPAD,), jnp.float32),
        pltpu.VMEM((_RPT, _GRID), jnp.float32),
        pltpu.VMEM((16,), jnp.float32),
        pltpu.VMEM((_NT * 16,), jnp.float32),
        pltpu.VMEM((_NT * _HPAD,), jnp.float32),
        pltpu.VMEM((_HPAD,), jnp.int32),
        pltpu.VMEM((80,), jnp.float32),
        pltpu.VMEM((80,), jnp.float32),
        pltpu.SemaphoreType.DMA,
        pltpu.VMEM_SHARED((_NT * _HPAD,), jnp.float32),
        pltpu.VMEM_SHARED((_NT * _HPAD,), jnp.float32),
        pltpu.VMEM_SHARED((_NT * _HPAD,), jnp.float32),
        pltpu.VMEM_SHARED((_NT * 80,), jnp.float32),
        pltpu.VMEM_SHARED((_NT * 16,), jnp.float32),
    ]


def _gauss_placement_matrix():
    c = np.arange(-(_PATCH // 2), _PATCH // 2 + 1, dtype=np.float32)
    e = np.exp(-(c ** 2) / (2.0 * _RADIUS ** 2)).astype(np.float32)
    g = (e / e.sum()).astype(np.float32)
    pos = np.maximum(np.arange(_GRID) * (_CANVAS // _GRID) - _PATCH // 2, 0)
    a = np.zeros((_CANVAS, _GRID), dtype=np.float32)
    for y in range(_GRID):
        a[pos[y]:pos[y] + _PATCH, y] = g
    return a


def _narrow_blocks(a):
    na = np.zeros((_NBLK, _BLK, _KW), dtype=np.float32)
    for i in range(_NBLK):
        b = min((_BLK // 8) * i, _GRID - _KW)
        na[i] = a[_BLK * i:_BLK * (i + 1), b:b + _KW]
    return na


def _render_kernel(mask_ref, at_ref, na_ref, out_ref, w2_ref):
    i = pl.program_id(0)

    @pl.when(i == 0)
    def _prep():
        mf = mask_ref[...]
        m = mf * (0.5 + _C1 * mf.T)
        w2_ref[...] = jax.lax.dot(m, at_ref[...])

    b = pl.multiple_of((_BLK // 8) * jnp.minimum(i, (_GRID - _KW) // (_BLK // 8)),
                       _BLK // 8)
    out_ref[...] = jax.lax.dot(na_ref[0], w2_ref[pl.ds(b, _KW), :])


@functools.partial(jax.jit, static_argnames=())
def kernel(logits, u, canvas):
    soft = pl.pallas_call(
        _softmax_kernel,
        out_shape=jax.ShapeDtypeStruct((_GRID, _GRID), jnp.float32),
    )(logits, u)

    mesh = plsc.VectorSubcoreMesh(core_axis_name="c", subcore_axis_name="s")
    mask = pl.kernel(
        _sc_body,
        out_type=jax.ShapeDtypeStruct((_GRID, _GRID), jnp.float32),
        mesh=mesh,
        scratch_types=_sc_scratch_types(),
    )(soft)

    a = _gauss_placement_matrix()
    at = jnp.asarray(np.ascontiguousarray(a.T))
    na = jnp.asarray(_narrow_blocks(a))
    out = pl.pallas_call(
        _render_kernel,
        grid=(_NBLK,),
        in_specs=[
            pl.BlockSpec((_GRID, _GRID), lambda i: (0, 0)),
            pl.BlockSpec((_GRID, _CANVAS), lambda i: (0, 0)),
            pl.BlockSpec((1, _BLK, _KW), lambda i: (i, 0, 0)),
        ],
        out_specs=pl.BlockSpec((_BLK, _CANVAS), lambda i: (i, 0)),
        out_shape=jax.ShapeDtypeStruct((_CANVAS, _CANVAS), jnp.float32),
        scratch_shapes=[pltpu.VMEM((_GRID, _CANVAS), jnp.float32)],
    )(mask, at, na)
    return out


# speedup vs baseline: 1.4849x; 1.4849x over previous
import functools

import numpy as np
import jax
import jax.numpy as jnp
from jax import lax
from jax.experimental import pallas as pl
from jax.experimental.pallas import tpu as pltpu
from jax.experimental.pallas import tpu_sc as plsc

_GRID = 256
_CANVAS = 2048
_NUM_DOTS = 4096
_PATCH = 15
_RADIUS = 2.0
_SIG1 = float(1.0 / (1.0 + np.exp(-1.0)))
_C1 = _SIG1 - 0.5
_BLK = 256
_NBLK = _CANVAS // _BLK
_KW = 64

_NT = 16
_RPT = _GRID // _NT
_EPT = _RPT * _GRID
_NROW = _EPT // 128
_NB = 1024
_HPAD = 1152


def _softmax_kernel(logits_ref, u_ref, soft_ref):
    g = -jnp.log(-jnp.log(u_ref[...]))
    y = logits_ref[...] + g
    y = y - jnp.max(y, axis=1, keepdims=True)
    ey = jnp.exp(y)
    soft_ref[...] = ey / jnp.sum(ey, axis=1, keepdims=True)


def _sc_iota16():
    return lax.broadcasted_iota(jnp.int32, (16,), 0)


def _hsum16(v):
    acc = v[0]
    for l in range(1, 16):
        acc = acc + v[l]
    return acc


def _sc_scan_hist(tmp_ref, csum_ref, need):
    above = jnp.float32(0.0)
    cid = jnp.int32(0)
    cabove = jnp.float32(0.0)
    for q in range(3, -1, -1):
        ch = csum_ref[16 * q:16 * (q + 1)]
        for l in range(15, -1, -1):
            v = ch[l]
            hit = jnp.logical_and(above < need, above + v >= need)
            cid = jnp.where(hit, 16 * q + l, cid)
            cabove = jnp.where(hit, above, cabove)
            above = above + v
    ch2 = tmp_ref[pl.ds(cid * 16, 16)]
    a = cabove
    b = jnp.int32(0)
    na = jnp.float32(0.0)
    cb = jnp.float32(0.0)
    for l in range(15, -1, -1):
        v = ch2[l]
        hit = jnp.logical_and(a < need, a + v >= need)
        b = jnp.where(hit, cid * 16 + l, b)
        na = jnp.where(hit, a, na)
        cb = jnp.where(hit, v, cb)
        a = a + v
    return b, na, cb


def _sc_body(soft_ref, mask_ref, vals_ref, keys_ref, ones_ref, tmp_ref,
             maskb_ref, stage_ref, pref16_ref, hist16_ref, cs_keys_ref,
             csum_ref, z80_ref, sem,
             sh_f1_ref, sh_f2_ref, sh_f3_ref, sh_cs_ref, sh_pref_ref):
    c = lax.axis_index("c")
    s = lax.axis_index("s")
    io16 = _sc_iota16()

    def ob(j, carry):
        ones_ref[pl.ds(j * 16, 16)] = jnp.ones((16,), jnp.float32)
        return carry

    lax.fori_loop(0, _EPT // 16, ob, 0)

    def ckb(j, carry):
        cs_keys_ref[pl.ds(j * 16, 16)] = (
            lax.shift_right_logical(j * 16 + io16, 4) + s * 80)
        return carry

    lax.fori_loop(0, _HPAD // 16, ckb, 0)
    for j in range(5):
        z80_ref[pl.ds(16 * j, 16)] = jnp.zeros((16,), jnp.float32)

    pltpu.sync_copy(soft_ref.at[pl.ds(s * _RPT, _RPT)], vals_ref)

    zeros = jnp.zeros((16,), jnp.float32)

    def zb(j, carry):
        tmp_ref[pl.ds(j * 16, 16)] = zeros
        return carry

    lax.fori_loop(0, _HPAD // 16, zb, 0)

    pltpu.sync_copy(tmp_ref, sh_f1_ref.at[pl.ds(s * _HPAD, _HPAD)])
    pltpu.sync_copy(tmp_ref, sh_f2_ref.at[pl.ds(s * _HPAD, _HPAD)])
    pltpu.sync_copy(tmp_ref, sh_f3_ref.at[pl.ds(s * _HPAD, _HPAD)])
    plsc.subcore_barrier()

    def bits_chunk(r, jj):
        return lax.bitcast_convert_type(vals_ref[r, 16 * jj:16 * (jj + 1)],
                                        jnp.int32)

    def do_level(sh_ref, key_fn):
        def kb(r, carry):
            for jj in range(16):
                i = r * 16 + jj
                keys_ref[pl.ds(i * 16, 16)] = (key_fn(bits_chunk(r, jj))
                                               + s * _HPAD)
            return carry

        lax.fori_loop(0, _RPT, kb, 0)

        pltpu.sync_copy(ones_ref, sh_ref.at[keys_ref], add=True)
        plsc.subcore_barrier()

        pltpu.sync_copy(sh_ref, hist16_ref)

        def fold(j, carry):
            acc = hist16_ref[pl.ds(j * 16, 16)]
            for t in range(1, _NT):
                acc = acc + hist16_ref[pl.ds(t * _HPAD + j * 16, 16)]
            tmp_ref[pl.ds(j * 16, 16)] = acc
            return carry

        lax.fori_loop(0, _HPAD // 16, fold, 0)

        pltpu.sync_copy(z80_ref, sh_cs_ref.at[pl.ds(s * 80, 80)])
        pltpu.sync_copy(tmp_ref, sh_cs_ref.at[cs_keys_ref], add=True)
        pltpu.sync_copy(sh_cs_ref.at[pl.ds(s * 80, 80)], csum_ref)

    need1 = jnp.float32(float(_NUM_DOTS))

    do_level(sh_f1_ref, lambda b: lax.shift_right_logical(b, 20))
    b1, na1, _ = _sc_scan_hist(tmp_ref, csum_ref, need1)
    need2 = need1 - na1

    def key2(b):
        k1 = lax.shift_right_logical(b, 20)
        k2 = jnp.bitwise_and(lax.shift_right_logical(b, 10), 1023)
        trash = _NB + jnp.bitwise_and(k2, 63) - jnp.bitwise_and(k2, 15)
        return jnp.where(k1 == b1, k2, trash + io16)

    do_level(sh_f2_ref, key2)
    b2, na2, _ = _sc_scan_hist(tmp_ref, csum_ref, need2)
    need3 = need2 - na2

    def key3(b):
        k1 = lax.shift_right_logical(b, 20)
        k2 = jnp.bitwise_and(lax.shift_right_logical(b, 10), 1023)
        k3 = jnp.bitwise_and(b, 1023)
        sel = jnp.logical_and(k1 == b1, k2 == b2)
        trash = _NB + jnp.bitwise_and(k3, 63) - jnp.bitwise_and(k3, 15)
        return jnp.where(sel, k3, trash + io16)

    do_level(sh_f3_ref, key3)
    b3, na3, n_eq_tot = _sc_scan_hist(tmp_ref, csum_ref, need3)
    n_eq_needed = need3 - na3
    vstar = jnp.bitwise_or(
        jnp.bitwise_or(lax.shift_left(b1, 20), lax.shift_left(b2, 10)), b3)

    ties = n_eq_tot > n_eq_needed

    def count_eq(_):
        def eq_cnt_body(r, acc):
            for jj in range(16):
                eqf = jnp.where(bits_chunk(r, jj) == vstar, 1.0, 0.0)
                acc = acc + _hsum16(eqf)
            return acc

        return lax.fori_loop(0, _RPT, eq_cnt_body, jnp.float32(0.0))

    eq_local = lax.cond(ties, count_eq, lambda _: jnp.float32(0.0), 0)

    stage_ref[...] = jnp.zeros((16,), jnp.float32) + eq_local
    pltpu.sync_copy(stage_ref, sh_pref_ref.at[pl.ds(s * 16, 16)])
    plsc.subcore_barrier()
    pltpu.sync_copy(sh_pref_ref, pref16_ref)
    my_base = jnp.float32(0.0)
    for t in range(_NT):
        cnt_t = pref16_ref[pl.ds(t * 16, 16)][0]
        my_base = my_base + jnp.where(t < s, cnt_t, 0.0)

    def mk_simple(_):
        def body(r, carry):
            for jj in range(16):
                sel = bits_chunk(r, jj) >= vstar
                maskb_ref[r, 16 * jj:16 * (jj + 1)] = jnp.where(
                    sel, 1.0, 0.0)
            return carry

        lax.fori_loop(0, _RPT, body, 0)
        return 0

    def mk_ties(_):
        def body(r, run):
            prefix = run
            for jj in range(16):
                b = bits_chunk(r, jj)
                eqf = jnp.where(b == vstar, 1.0, 0.0)
                out = jnp.where(b > vstar, 1.0, 0.0)
                for l in range(16):
                    e = eqf[l]
                    takef = jnp.where(
                        jnp.logical_and(e > 0.5, prefix < n_eq_needed),
                        1.0, 0.0)
                    out = out + jnp.where(io16 == l, takef, 0.0)
                    prefix = prefix + e
                maskb_ref[r, 16 * jj:16 * (jj + 1)] = out
            return prefix

        lax.fori_loop(0, _RPT, body, my_base)
        return 0

    lax.cond(ties, mk_ties, mk_simple, 0)

    @pl.when(c == 0)
    def _write():
        pltpu.sync_copy(maskb_ref, mask_ref.at[pl.ds(s * _RPT, _RPT)])


def _sc_scratch_types():
    return [
        pltpu.VMEM((_RPT, _GRID), jnp.float32),
        pltpu.VMEM((_EPT,), jnp.int32),
        pltpu.VMEM((_EPT,), jnp.float32),
        pltpu.VMEM((_HPAD,), jnp.float32),
        pltpu.VMEM((_RPT, _GRID), jnp.float32),
        pltpu.VMEM((16,), jnp.float32),
        pltpu.VMEM((_NT * 16,), jnp.float32),
        pltpu.VMEM((_NT * _HPAD,), jnp.float32),
        pltpu.VMEM((_HPAD,), jnp.int32),
        pltpu.VMEM((80,), jnp.float32),
        pltpu.VMEM((80,), jnp.float32),
        pltpu.SemaphoreType.DMA,
        pltpu.VMEM_SHARED((_NT * _HPAD,), jnp.float32),
        pltpu.VMEM_SHARED((_NT * _HPAD,), jnp.float32),
        pltpu.VMEM_SHARED((_NT * _HPAD,), jnp.float32),
        pltpu.VMEM_SHARED((_NT * 80,), jnp.float32),
        pltpu.VMEM_SHARED((_NT * 16,), jnp.float32),
    ]


def _gauss_placement_matrix():
    c = np.arange(-(_PATCH // 2), _PATCH // 2 + 1, dtype=np.float32)
    e = np.exp(-(c ** 2) / (2.0 * _RADIUS ** 2)).astype(np.float32)
    g = (e / e.sum()).astype(np.float32)
    pos = np.maximum(np.arange(_GRID) * (_CANVAS // _GRID) - _PATCH // 2, 0)
    a = np.zeros((_CANVAS, _GRID), dtype=np.float32)
    for y in range(_GRID):
        a[pos[y]:pos[y] + _PATCH, y] = g
    return a


def _narrow_blocks(a):
    na = np.zeros((_NBLK, _BLK, _KW), dtype=np.float32)
    for i in range(_NBLK):
        b = min((_BLK // 8) * i, _GRID - _KW)
        na[i] = a[_BLK * i:_BLK * (i + 1), b:b + _KW]
    return na


def _render_kernel(mask_ref, at_ref, na_ref, out_ref, w2_ref):
    i = pl.program_id(0)

    @pl.when(i == 0)
    def _prep():
        mf = mask_ref[...]
        m = mf * (0.5 + _C1 * mf.T)
        w2_ref[...] = jax.lax.dot(m, at_ref[...])

    b = pl.multiple_of((_BLK // 8) * jnp.minimum(i, (_GRID - _KW) // (_BLK // 8)),
                       _BLK // 8)
    out_ref[...] = jax.lax.dot(na_ref[0], w2_ref[pl.ds(b, _KW), :])


@functools.partial(jax.jit, static_argnames=())
def kernel(logits, u, canvas):
    soft = pl.pallas_call(
        _softmax_kernel,
        out_shape=jax.ShapeDtypeStruct((_GRID, _GRID), jnp.float32),
    )(logits, u)

    mesh = plsc.VectorSubcoreMesh(core_axis_name="c", subcore_axis_name="s")
    mask = pl.kernel(
        _sc_body,
        out_type=jax.ShapeDtypeStruct((_GRID, _GRID), jnp.float32),
        mesh=mesh,
        scratch_types=_sc_scratch_types(),
    )(soft)

    a = _gauss_placement_matrix()
    at = jnp.asarray(np.ascontiguousarray(a.T))
    na = jnp.asarray(_narrow_blocks(a))
    out = pl.pallas_call(
        _render_kernel,
        grid=(_NBLK,),
        in_specs=[
            pl.BlockSpec((_GRID, _GRID), lambda i: (0, 0)),
            pl.BlockSpec((_GRID, _CANVAS), lambda i: (0, 0)),
            pl.BlockSpec((1, _BLK, _KW), lambda i: (i, 0, 0)),
        ],
        out_specs=pl.BlockSpec((_BLK, _CANVAS), lambda i: (i, 0)),
        out_shape=jax.ShapeDtypeStruct((_CANVAS, _CANVAS), jnp.float32),
        scratch_shapes=[pltpu.VMEM((_GRID, _CANVAS), jnp.float32)],
    )(mask, at, na)
    return out
